# baseline (speedup 1.0000x reference)
import functools
import jax
import jax.numpy as jnp
from jax import lax
from jax.experimental import pallas as pl
from jax.experimental.pallas import tpu as pltpu
from jax.experimental.pallas import tpu_sc as plsc

B = 8192
T = 65536
D = 192
INNER = 384
NUM = 14
TILE = 256
GRID = B // TILE

VP_MOV = 136
VP_CAP = 136
VP_EV = 72
VP_COND = 136
VP_EFF = 136
VP_SN = 72
VP_ST = 8


def _gelu(x):
    return 0.5 * x * (1.0 + jax.lax.erf(x / jnp.sqrt(2.0).astype(x.dtype)))


def _tc_body(bid_ref, nf_ref,
             cm_ref, cc_ref, ce_ref, ccd_ref, cef_ref, csn_ref, cst_ref,
             bemb_ref, tm_ref, tc_ref, te_ref, tcd_ref, tef_ref, tsn_ref, tst_ref,
             w1_ref, b1_ref, w2_ref, b2_ref, lng_ref, lnb_ref,
             wo1_ref, bo1_ref, wo2_ref, bo2_ref,
             out_ref):
    f32 = jnp.float32

    def msum(cref, tref):
        c2 = cref[...]
        v = tref.shape[0]
        c = c2[0, :, :v] + c2[1, :, :v]
        return jnp.dot(c, tref[...], preferred_element_type=f32)

    pooled = msum(cm_ref, tm_ref)
    pooled += msum(cc_ref, tc_ref)
    pooled += msum(ce_ref, te_ref)
    pooled += msum(ccd_ref, tcd_ref)
    pooled += msum(cef_ref, tef_ref)
    pooled += msum(csn_ref, tsn_ref)
    cst2 = cst_ref[...]
    cst = cst2[0, :, :7] + cst2[1, :, :7]
    pooled += jnp.dot(cst, tst_ref[...], preferred_element_type=f32)

    c = jnp.sum(cst, axis=-1, keepdims=True)
    pooled = jnp.where(c > 0.0, pooled / jnp.maximum(c, 1.0), 0.0)

    bid = bid_ref[0, 0, :]
    oh = (bid[:, None] == jax.lax.broadcasted_iota(jnp.int32, (TILE, 9), 1))
    base = jnp.dot(oh.astype(f32), bemb_ref[...], preferred_element_type=f32)

    nf = nf_ref[...]
    h1 = _gelu(jnp.dot(nf, w1_ref[...], preferred_element_type=f32) + b1_ref[...])
    num = jnp.dot(h1, w2_ref[...], preferred_element_type=f32) + b2_ref[...]

    combined = base + pooled + num

    m = jnp.mean(combined, axis=-1, keepdims=True)
    v = jnp.mean((combined - m) ** 2, axis=-1, keepdims=True)
    h = (combined - m) * jax.lax.rsqrt(v + 1e-5) * lng_ref[...] + lnb_ref[...]

    h2 = _gelu(jnp.dot(h, wo1_ref[...], preferred_element_type=f32) + bo1_ref[...])
    out_ref[...] = jnp.dot(h2, wo2_ref[...], preferred_element_type=f32) + bo2_ref[...]


def _tile_spec(vp):
    return pl.BlockSpec((2, TILE, vp), lambda i: (0, i, 0))


def _full_spec(shape):
    nd = len(shape)
    return pl.BlockSpec(shape, lambda i, _n=nd: (0,) * _n)


@jax.jit
def _tc_fused(base_ids, numeric, counts, bemb_p, tables_p,
              W1, b1, W2, b2, ln_g, ln_b, Wo1, bo1, Wo2, bo2):
    cm, cc, ce, ccd, cef, csn, cst = counts
    tm, tcap, te, tcd, tef, tsn, tst = tables_p
    bid3 = base_ids.reshape(GRID, 1, TILE)
    in_specs = [
        pl.BlockSpec((1, 1, TILE), lambda i: (i, 0, 0)),
        pl.BlockSpec((TILE, NUM), lambda i: (i, 0)),
        _tile_spec(VP_MOV), _tile_spec(VP_CAP), _tile_spec(VP_EV),
        _tile_spec(VP_COND), _tile_spec(VP_EFF), _tile_spec(VP_SN),
        _tile_spec(VP_ST),
        _full_spec((9, D)),
        _full_spec((129, D)), _full_spec((129, D)), _full_spec((65, D)),
        _full_spec((129, D)), _full_spec((129, D)), _full_spec((65, D)),
        _full_spec((7, D)),
        _full_spec((NUM, INNER)), _full_spec((INNER,)),
        _full_spec((INNER, D)), _full_spec((D,)),
        _full_spec((D,)), _full_spec((D,)),
        _full_spec((D, INNER)), _full_spec((INNER,)),
        _full_spec((INNER, D)), _full_spec((D,)),
    ]
    return pl.pallas_call(
        _tc_body,
        grid=(GRID,),
        in_specs=in_specs,
        out_specs=pl.BlockSpec((TILE, D), lambda i: (i, 0)),
        out_shape=jax.ShapeDtypeStruct((B, D), jnp.float32),
    )(bid3, numeric, cm, cc, ce, ccd, cef, csn, cst, bemb_p,
      tm, tcap, te, tcd, tef, tsn, tst,
      W1, b1, W2, b2, ln_g, ln_b, Wo1, bo1, Wo2, bo2)


def _counts_xla(seg, ids, vp):
    out = jnp.zeros((2, B, vp), jnp.float32)
    return out.at[0, seg, ids].add(1.0)


NC = 2
NS = 16
NW = NC * NS
CHUNK = T // NW
SEGS_PER_SUB = B // NS
_VPS = (VP_MOV, VP_CAP, VP_EV, VP_COND, VP_EFF, VP_SN, VP_ST)
_ZWORDS = SEGS_PER_SUB * 8


def _sc_hist_body(seg_hbm, m_hbm, c_hbm, e_hbm, cd_hbm, ef_hbm, sn_hbm, st_hbm,
                  o_m, o_c, o_e, o_cd, o_ef, o_sn, o_st,
                  shared, seg_v, ids_v, idx_v, ones_v, zeros_v):
    cid = lax.axis_index("c")
    sid = lax.axis_index("s")
    wid = cid * NS + sid
    base = wid * CHUNK

    zero16 = jnp.zeros((16,), jnp.float32)

    def zinit(i, carry):
        zeros_v[pl.ds(i * 16, 16)] = zero16
        return carry

    lax.fori_loop(0, _ZWORDS // 16, zinit, 0)
    for k in range(8):
        ones_v[pl.ds(k * 16, 16)] = jnp.ones((16,), jnp.float32)

    pltpu.sync_copy(seg_hbm.at[pl.ds(base, CHUNK)], seg_v)

    for ids_hbm, out_ref, W in zip(
            (m_hbm, c_hbm, e_hbm, cd_hbm, ef_hbm, sn_hbm, st_hbm),
            (o_m, o_c, o_e, o_cd, o_ef, o_sn, o_st), _VPS):
        stripe = SEGS_PER_SUB * W

        def zbody(i, carry, _stripe=stripe):
            pltpu.sync_copy(zeros_v,
                            shared.at[pl.ds(sid * _stripe + i * _ZWORDS, _ZWORDS)])
            return carry

        lax.fori_loop(0, stripe // _ZWORDS, zbody, 0)
        pltpu.sync_copy(ids_hbm.at[pl.ds(base, CHUNK)], ids_v)
        plsc.subcore_barrier()

        def jbody(j, carry, _W=W):
            for k in range(8):
                o = j * 128 + k * 16
                s16 = seg_v[pl.ds(o, 16)]
                i16 = ids_v[pl.ds(o, 16)]
                idx_v[j, pl.ds(k * 16, 16)] = s16 * _W + i16
            return carry

        lax.fori_loop(0, 16, jbody, 0)

        def sbody(j, carry):
            pltpu.sync_copy(ones_v, shared.at[idx_v.at[j]], add=True)
            return carry

        lax.fori_loop(0, 16, sbody, 0)
        plsc.subcore_barrier()
        pltpu.sync_copy(shared.at[pl.ds(sid * stripe, stripe)],
                        out_ref.at[cid, pl.ds(sid * stripe, stripe)])
        plsc.subcore_barrier()


@jax.jit
def _sc_hist(seg, m, c, e, cd, ef, sn, st):
    mesh = plsc.VectorSubcoreMesh(core_axis_name="c", subcore_axis_name="s")
    f = pl.kernel(
        _sc_hist_body,
        out_type=[jax.ShapeDtypeStruct((NC, B * W), jnp.float32) for W in _VPS],
        mesh=mesh,
        scratch_types=[
            pltpu.VMEM_SHARED((B * VP_MOV,), jnp.float32),
            pltpu.VMEM((CHUNK,), jnp.int32),
            pltpu.VMEM((CHUNK,), jnp.int32),
            pltpu.VMEM((16, 128), jnp.int32),
            pltpu.VMEM((128,), jnp.float32),
            pltpu.VMEM((_ZWORDS,), jnp.float32),
        ],
    )
    outs = f(seg, m, c, e, cd, ef, sn, st)
    return tuple(o.reshape(NC, B, W) for o, W in zip(outs, _VPS))


def _pad_rows(t, rows):
    return jnp.zeros((rows, t.shape[1]), t.dtype).at[: t.shape[0]].set(t)


def kernel(base_ids, movement_ids, capture_ids, hook_event_ids, condition_ids,
           effect_ids, state_name_ids, state_type_ids, segment_ids,
           numeric_features, base_emb, movement_emb, capture_emb, event_emb,
           condition_emb, effect_emb, sname_emb, stype_emb,
           W1, b1, W2, b2, ln_g, ln_b, Wo1, bo1, Wo2, bo2):
    i32 = jnp.int32
    seg = segment_ids.astype(i32)
    counts = _sc_hist(seg, movement_ids.astype(i32), capture_ids.astype(i32),
                      hook_event_ids.astype(i32), condition_ids.astype(i32),
                      effect_ids.astype(i32), state_name_ids.astype(i32),
                      state_type_ids.astype(i32))
    tables = (movement_emb, capture_emb, event_emb, condition_emb,
              effect_emb, sname_emb, stype_emb)
    return _tc_fused(base_ids.astype(i32), numeric_features, counts, base_emb,
                     tables, W1, b1, W2, b2, ln_g, ln_b, Wo1, bo1, Wo2, bo2)


# speedup vs baseline: 2.2392x; 2.2392x over previous
import jax
import jax.numpy as jnp
from jax import lax
from jax.experimental import pallas as pl
from jax.experimental.pallas import tpu as pltpu
from jax.experimental.pallas import tpu_sc as plsc

B = 8192
T = 65536
D = 192
INNER = 384
NUM = 14
TILE = 256
GRID = B // TILE
W = 128

ST_OFF = 65


def _gelu(x):
    return 0.5 * x * (1.0 + jax.lax.erf(x / jnp.sqrt(2.0).astype(x.dtype)))


def _tc_body(bid_ref, nf_ref,
             cm_ref, cc_ref, ccd_ref, cef_ref, c5_ref, csn_ref,
             bemb_ref, tm_ref, tc_ref, tcd_ref, tef_ref, t5_ref, tsn_ref,
             w1_ref, b1_ref, w2_ref, b2_ref, lng_ref, lnb_ref,
             wo1_ref, bo1_ref, wo2_ref, bo2_ref,
             out_ref):
    f32 = jnp.float32

    s5 = c5_ref[0] + c5_ref[1]
    c = jnp.sum(s5[:, ST_OFF:ST_OFF + 7], axis=-1, keepdims=True)

    pooled = jnp.dot(s5, t5_ref[...], preferred_element_type=f32)
    sn = csn_ref[0] + csn_ref[1]
    pooled += jnp.dot(sn, tsn_ref[...], preferred_element_type=f32)

    def msum129(cref, tref):
        cs = cref[0] + cref[1]
        main = jnp.dot(cs, tref[0:128, :], preferred_element_type=f32)
        over = c - jnp.sum(cs, axis=-1, keepdims=True)
        return main + over * tref[128:129, :]

    pooled += msum129(cm_ref, tm_ref)
    pooled += msum129(cc_ref, tc_ref)
    pooled += msum129(ccd_ref, tcd_ref)
    pooled += msum129(cef_ref, tef_ref)

    pooled = jnp.where(c > 0.0, pooled / jnp.maximum(c, 1.0), 0.0)

    bid = bid_ref[0, 0, :]
    oh = (bid[:, None] == jax.lax.broadcasted_iota(jnp.int32, (TILE, 9), 1))
    base = jnp.dot(oh.astype(f32), bemb_ref[...], preferred_element_type=f32)

    nf = nf_ref[...]
    h1 = _gelu(jnp.dot(nf, w1_ref[...], preferred_element_type=f32) + b1_ref[...])
    num = jnp.dot(h1, w2_ref[...], preferred_element_type=f32) + b2_ref[...]

    combined = base + pooled + num

    m = jnp.mean(combined, axis=-1, keepdims=True)
    v = jnp.mean((combined - m) ** 2, axis=-1, keepdims=True)
    h = (combined - m) * jax.lax.rsqrt(v + 1e-5) * lng_ref[...] + lnb_ref[...]

    h2 = _gelu(jnp.dot(h, wo1_ref[...], preferred_element_type=f32) + bo1_ref[...])
    out_ref[...] = jnp.dot(h2, wo2_ref[...], preferred_element_type=f32) + bo2_ref[...]


def _tile_spec():
    return pl.BlockSpec((2, TILE, W), lambda i: (0, i, 0))


def _full_spec(shape):
    nd = len(shape)
    return pl.BlockSpec(shape, lambda i, _n=nd: (0,) * _n)


@jax.jit
def _tc_fused(base_ids, numeric, counts, bemb_p, tm, tcap, tcd, tef, t5, tsn,
              W1, b1, W2, b2, ln_g, ln_b, Wo1, bo1, Wo2, bo2):
    cm, cc, ccd, cef, c5, csn = counts
    bid3 = base_ids.reshape(GRID, 1, TILE)
    in_specs = [
        pl.BlockSpec((1, 1, TILE), lambda i: (i, 0, 0)),
        pl.BlockSpec((TILE, NUM), lambda i: (i, 0)),
        _tile_spec(), _tile_spec(), _tile_spec(),
        _tile_spec(), _tile_spec(), _tile_spec(),
        _full_spec((9, D)),
        _full_spec((129, D)), _full_spec((129, D)), _full_spec((129, D)),
        _full_spec((129, D)),
        _full_spec((W, D)), _full_spec((W, D)),
        _full_spec((NUM, INNER)), _full_spec((INNER,)),
        _full_spec((INNER, D)), _full_spec((D,)),
        _full_spec((D,)), _full_spec((D,)),
        _full_spec((D, INNER)), _full_spec((INNER,)),
        _full_spec((INNER, D)), _full_spec((D,)),
    ]
    return pl.pallas_call(
        _tc_body,
        grid=(GRID,),
        in_specs=in_specs,
        out_specs=pl.BlockSpec((TILE, D), lambda i: (i, 0)),
        out_shape=jax.ShapeDtypeStruct((B, D), jnp.float32),
    )(bid3, numeric, cm, cc, ccd, cef, c5, csn, bemb_p,
      tm, tcap, tcd, tef, t5, tsn,
      W1, b1, W2, b2, ln_g, ln_b, Wo1, bo1, Wo2, bo2)


NC = 2
NS = 16
NW = NC * NS
CHUNK = T // NW
SEGS_PER_SUB = B // NS
STRIPE = SEGS_PER_SUB * W
GARB = B * W
_ZWORDS = 4096


def _sc_hist_body(seg_hbm, m_hbm, c_hbm, cd_hbm, ef_hbm, e_hbm, st_hbm, sn_hbm,
                  o_m, o_c, o_cd, o_ef, o_5, o_sn,
                  shared, seg_v, ids_v, idx_v, ones_v, zeros_v):
    cid = lax.axis_index("c")
    sid = lax.axis_index("s")
    wid = cid * NS + sid
    base = wid * CHUNK

    zero16 = jnp.zeros((16,), jnp.float32)

    def zinit(i, carry):
        zeros_v[pl.ds(i * 16, 16)] = zero16
        return carry

    lax.fori_loop(0, _ZWORDS // 16, zinit, 0)
    for k in range(8):
        ones_v[pl.ds(k * 16, 16)] = jnp.ones((16,), jnp.float32)

    pltpu.sync_copy(seg_hbm.at[pl.ds(base, CHUNK)], seg_v)

    passes = (
        ((m_hbm, 0, True),), ((c_hbm, 0, True),), ((cd_hbm, 0, True),),
        ((ef_hbm, 0, True),),
        ((e_hbm, 0, False), (st_hbm, ST_OFF, False)),
        ((sn_hbm, 0, False),),
    )
    outs = (o_m, o_c, o_cd, o_ef, o_5, o_sn)

    for sub, out_ref in zip(passes, outs):
        def zbody(i, carry):
            pltpu.sync_copy(zeros_v,
                            shared.at[pl.ds(sid * STRIPE + i * _ZWORDS, _ZWORDS)])
            return carry

        lax.fori_loop(0, STRIPE // _ZWORDS, zbody, 0)
        plsc.subcore_barrier()

        for ids_hbm, off, ovf in sub:
            pltpu.sync_copy(ids_hbm.at[pl.ds(base, CHUNK)], ids_v)

            def jbody(j, carry, _off=off, _ovf=ovf):
                for k in range(8):
                    o = j * 128 + k * 16
                    s16 = seg_v[pl.ds(o, 16)]
                    i16 = ids_v[pl.ds(o, 16)]
                    flat = s16 * W + (i16 + _off)
                    if _ovf:
                        flat = jnp.where(i16 < W, flat, GARB)
                    idx_v[j, pl.ds(k * 16, 16)] = flat
                return carry

            lax.fori_loop(0, 16, jbody, 0)

            def sbody(j, carry):
                pltpu.sync_copy(ones_v, shared.at[idx_v.at[j]], add=True)
                return carry

            lax.fori_loop(0, 16, sbody, 0)

        plsc.subcore_barrier()
        pltpu.sync_copy(shared.at[pl.ds(sid * STRIPE, STRIPE)],
                        out_ref.at[pl.ds(cid * (B * W) + sid * STRIPE, STRIPE)])


@jax.jit
def _sc_hist(seg, m, c, cd, ef, e, st, sn):
    mesh = plsc.VectorSubcoreMesh(core_axis_name="c", subcore_axis_name="s")
    f = pl.kernel(
        _sc_hist_body,
        out_type=[jax.ShapeDtypeStruct((NC * B * W,), jnp.float32)
                  for _ in range(6)],
        mesh=mesh,
        scratch_types=[
            pltpu.VMEM_SHARED((B * W + 128,), jnp.float32),
            pltpu.VMEM((CHUNK,), jnp.int32),
            pltpu.VMEM((CHUNK,), jnp.int32),
            pltpu.VMEM((16, 128), jnp.int32),
            pltpu.VMEM((128,), jnp.float32),
            pltpu.VMEM((_ZWORDS,), jnp.float32),
        ],
    )
    outs = f(seg, m, c, cd, ef, e, st, sn)
    return tuple(o.reshape(NC, B, W) for o in outs)


def kernel(base_ids, movement_ids, capture_ids, hook_event_ids, condition_ids,
           effect_ids, state_name_ids, state_type_ids, segment_ids,
           numeric_features, base_emb, movement_emb, capture_emb, event_emb,
           condition_emb, effect_emb, sname_emb, stype_emb,
           W1, b1, W2, b2, ln_g, ln_b, Wo1, bo1, Wo2, bo2):
    i32 = jnp.int32
    seg = segment_ids.astype(i32)
    counts = _sc_hist(seg, movement_ids.astype(i32), capture_ids.astype(i32),
                      condition_ids.astype(i32), effect_ids.astype(i32),
                      hook_event_ids.astype(i32), state_type_ids.astype(i32),
                      state_name_ids.astype(i32))
    t5 = jnp.zeros((W, D), jnp.float32).at[0:65].set(event_emb) \
        .at[ST_OFF:ST_OFF + 7].set(stype_emb)
    tsn = jnp.zeros((W, D), jnp.float32).at[0:65].set(sname_emb)
    return _tc_fused(base_ids.astype(i32), numeric_features, counts, base_emb,
                     movement_emb, capture_emb, condition_emb, effect_emb,
                     t5, tsn, W1, b1, W2, b2, ln_g, ln_b, Wo1, bo1, Wo2, bo2)
